# baseline (speedup 1.0000x reference)
import functools

import jax
import jax.numpy as jnp
from jax import lax
from jax.experimental import pallas as pl
from jax.experimental.pallas import tpu as pltpu
from jax.experimental.pallas import tpu_sc as plsc

NC = 2
NS = 16
NW = NC * NS
CHUNK = 80
D = 48
EPS = 1e-8

BLK = 2000


def _sc_gather(table, idx_flat):
    T = idx_flat.shape[0]
    per_w = T // NW
    n_chunks = per_w // CHUNK
    assert per_w * NW == T and n_chunks * CHUNK == per_w

    mesh = plsc.VectorSubcoreMesh(core_axis_name="c", subcore_axis_name="s")

    @functools.partial(
        pl.kernel,
        out_type=jax.ShapeDtypeStruct((T, D), jnp.float32),
        mesh=mesh,
        scratch_types=[
            pltpu.VMEM((CHUNK,), jnp.int32),
            pltpu.VMEM((CHUNK, D), jnp.float32),
            pltpu.SemaphoreType.DMA,
        ],
        compiler_params=pltpu.CompilerParams(use_tc_tiling_on_sc=False),
    )
    def k(idx_hbm, table_hbm, out_hbm, idx_v, rows_v, sem):
        wid = lax.axis_index("s") * NC + lax.axis_index("c")
        base = wid * per_w

        def body(i, carry):
            st = base + i * CHUNK
            pltpu.sync_copy(idx_hbm.at[pl.ds(st, CHUNK)], idx_v)
            pltpu.async_copy(table_hbm.at[idx_v], rows_v, sem).wait()
            pltpu.sync_copy(rows_v, out_hbm.at[pl.ds(st, CHUNK)])
            return carry

        lax.fori_loop(0, n_chunks, body, 0)

    return k(idx_flat, table)


def _norm2d(x2, eps=EPS):
    return jnp.sqrt(jnp.maximum(x2, eps))


def _tc_body(gsrc, gdst, hes, hev,
             m1_whT, m1_wsT, m1_b, m1_wv,
             m2_wsT, m2_b, m3_wsT, m3_b, scal,
             ln0_g, ln0_b,
             f1_whT, f1_wsT, f1_b, f1_wvT,
             f2_whT, f2_wsT, f2_b, f2_wvT,
             ln1_g, ln1_b,
             xs_out, xv_out):
    gs = gsrc[...]
    gd = gdst[...]
    s_i = gs[:, 0:32]
    s_j = gd[:, 0:32]
    he_s = hes[...]
    he_v = hev[...]
    vex = he_v[:, 0:1]
    vey = he_v[:, 1:2]
    vez = he_v[:, 2:3]

    vx = jnp.concatenate([gs[:, 32:36], vex, gd[:, 32:36]], axis=1)
    vy = jnp.concatenate([gs[:, 36:40], vey, gd[:, 36:40]], axis=1)
    vz = jnp.concatenate([gs[:, 40:44], vez, gd[:, 40:44]], axis=1)

    f32 = jnp.float32

    def dot(a, b):
        return jax.lax.dot_general(a, b, (((1,), (0,)), ((), ())),
                                   preferred_element_type=f32)

    w = m1_whT[...]
    vhx = dot(vx, w)
    vhy = dot(vy, w)
    vhz = dot(vz, w)
    vn = _norm2d(vhx * vhx + vhy * vhy + vhz * vhz)
    s_cat = jnp.concatenate([s_i, he_s, s_j, vn], axis=1)
    s = dot(s_cat, m1_wsT[...]) + m1_b[...]
    wv = m1_wv[...]
    vox = jnp.sum(vhx * wv, axis=1, keepdims=True)
    voy = jnp.sum(vhy * wv, axis=1, keepdims=True)
    voz = jnp.sum(vhz * wv, axis=1, keepdims=True)
    gate = jax.nn.sigmoid(_norm2d(vox * vox + voy * voy + voz * voz))
    vox, voy, voz = vox * gate, voy * gate, voz * gate
    s = jnp.maximum(s, 0.0)

    m2_wh = scal[0, 0]
    m2_wv = scal[0, 1]
    vhx, vhy, vhz = vox * m2_wh, voy * m2_wh, voz * m2_wh
    vn = _norm2d(vhx * vhx + vhy * vhy + vhz * vhz)
    s = dot(jnp.concatenate([s, vn], axis=1), m2_wsT[...]) + m2_b[...]
    vox, voy, voz = vhx * m2_wv, vhy * m2_wv, vhz * m2_wv
    gate = jax.nn.sigmoid(_norm2d(vox * vox + voy * voy + voz * voz))
    vox, voy, voz = vox * gate, voy * gate, voz * gate
    s = jnp.maximum(s, 0.0)

    m3_wh = scal[0, 2]
    m3_wv = scal[0, 3]
    vhx, vhy, vhz = vox * m3_wh, voy * m3_wh, voz * m3_wh
    vn = _norm2d(vhx * vhx + vhy * vhy + vhz * vhz)
    s = dot(jnp.concatenate([s, vn], axis=1), m3_wsT[...]) + m3_b[...]
    vox, voy, voz = vhx * m3_wv, vhy * m3_wv, vhz * m3_wv

    xs = he_s + s
    xvx, xvy, xvz = vex + vox, vey + voy, vez + voz

    def gvp_ln(xs, xvx, xvy, xvz, g, b):
        vn2 = jnp.maximum(xvx * xvx + xvy * xvy + xvz * xvz, EPS)
        denom = jnp.sqrt(vn2)
        xvx, xvy, xvz = xvx / denom, xvy / denom, xvz / denom
        mu = jnp.mean(xs, axis=1, keepdims=True)
        var = jnp.mean(jnp.square(xs - mu), axis=1, keepdims=True)
        xs = (xs - mu) / jnp.sqrt(var + 1e-5) * g[...] + b[...]
        return xs, xvx, xvy, xvz

    xs, xvx, xvy, xvz = gvp_ln(xs, xvx, xvy, xvz, ln0_g, ln0_b)

    whT = f1_whT[...]
    vhx = xvx * whT
    vhy = xvy * whT
    vhz = xvz * whT
    vn = _norm2d(vhx * vhx + vhy * vhy + vhz * vhz)
    ds = dot(jnp.concatenate([xs, vn], axis=1), f1_wsT[...]) + f1_b[...]
    wvT = f1_wvT[...]
    vox = vhx[:, 0:1] * wvT[0:1, :] + vhx[:, 1:2] * wvT[1:2, :]
    voy = vhy[:, 0:1] * wvT[0:1, :] + vhy[:, 1:2] * wvT[1:2, :]
    voz = vhz[:, 0:1] * wvT[0:1, :] + vhz[:, 1:2] * wvT[1:2, :]
    gate = jax.nn.sigmoid(_norm2d(vox * vox + voy * voy + voz * voz))
    vox, voy, voz = vox * gate, voy * gate, voz * gate
    ds = jnp.maximum(ds, 0.0)

    whT = f2_whT[...]
    vhx = vox[:, 0:1] * whT[0:1, :] + vox[:, 1:2] * whT[1:2, :]
    vhy = voy[:, 0:1] * whT[0:1, :] + voy[:, 1:2] * whT[1:2, :]
    vhz = voz[:, 0:1] * whT[0:1, :] + voz[:, 1:2] * whT[1:2, :]
    vn = _norm2d(vhx * vhx + vhy * vhy + vhz * vhz)
    ds = dot(jnp.concatenate([ds, vn], axis=1), f2_wsT[...]) + f2_b[...]
    wvT = f2_wvT[...]
    vox = vhx[:, 0:1] * wvT[0, 0] + vhx[:, 1:2] * wvT[1, 0]
    voy = vhy[:, 0:1] * wvT[0, 0] + vhy[:, 1:2] * wvT[1, 0]
    voz = vhz[:, 0:1] * wvT[0, 0] + vhz[:, 1:2] * wvT[1, 0]

    xs = xs + ds
    xvx, xvy, xvz = xvx + vox, xvy + voy, xvz + voz
    xs, xvx, xvy, xvz = gvp_ln(xs, xvx, xvy, xvz, ln1_g, ln1_b)

    xs_out[...] = xs
    xv_out[...] = jnp.concatenate([xvx, xvy, xvz], axis=1)


def kernel(h_V_s, h_V_v, edge_index, h_E_s, h_E_v, params):
    p = params
    N, si = h_V_s.shape
    E = edge_index.shape[1]
    f32 = jnp.float32

    table = jnp.concatenate(
        [h_V_s, h_V_v[:, :, 0], h_V_v[:, :, 1], h_V_v[:, :, 2],
         jnp.zeros((N, D - si - 12), f32)], axis=1)

    idx_flat = edge_index.reshape(-1)
    g_all = _sc_gather(table, idx_flat)

    hev = h_E_v.reshape(E, 3)

    n_blk = E // BLK
    assert n_blk * BLK == E

    def bspec(shape):
        return pl.BlockSpec(shape, lambda i: (i, 0))

    def wspec(a):
        return pl.BlockSpec(a.shape, lambda i: tuple(0 for _ in a.shape))

    weights = [
        p['m1_wh'].T, p['m1_ws_w'].T, p['m1_ws_b'].reshape(1, 32), p['m1_wv'],
        p['m2_ws_w'].T, p['m2_ws_b'].reshape(1, 32),
        p['m3_ws_w'].T, p['m3_ws_b'].reshape(1, 32),
        jnp.stack([p['m2_wh'][0, 0], p['m2_wv'][0, 0],
                   p['m3_wh'][0, 0], p['m3_wv'][0, 0]]).reshape(1, 4),
        p['ln0_g'].reshape(1, 32), p['ln0_b'].reshape(1, 32),
        p['f1_wh'].T, p['f1_ws_w'].T, p['f1_ws_b'].reshape(1, 128), p['f1_wv'].T,
        p['f2_wh'].T, p['f2_ws_w'].T, p['f2_ws_b'].reshape(1, 32), p['f2_wv'].T,
        p['ln1_g'].reshape(1, 32), p['ln1_b'].reshape(1, 32),
    ]

    in_specs = ([pl.BlockSpec((BLK, D), lambda i: (i, 0)),
                 pl.BlockSpec((BLK, D), lambda i: (i + n_blk, 0)),
                 bspec((BLK, si)), bspec((BLK, 3))]
                + [wspec(w) for w in weights])

    xs, xv = pl.pallas_call(
        _tc_body,
        grid=(n_blk,),
        in_specs=in_specs,
        out_specs=[bspec((BLK, si)), bspec((BLK, 3))],
        out_shape=[jax.ShapeDtypeStruct((E, si), f32),
                   jax.ShapeDtypeStruct((E, 3), f32)],
    )(g_all, g_all, h_E_s, hev, *weights)

    return xs, xv.reshape(E, 1, 3)


# speedup vs baseline: 1.0085x; 1.0085x over previous
import functools

import jax
import jax.numpy as jnp
from jax import lax
from jax.experimental import pallas as pl
from jax.experimental.pallas import tpu as pltpu
from jax.experimental.pallas import tpu_sc as plsc

NC = 2
NS = 16
NW = NC * NS
CHUNK = 128
D = 64
EPS = 1e-8

BLK = 2000


def _sc_gather(table, idx_flat):
    T = idx_flat.shape[0]
    per_w = T // NW
    assert per_w * NW == T and per_w % 8 == 0 and (per_w - CHUNK) % 8 == 0
    n_chunks = (per_w + CHUNK - 1) // CHUNK
    n_pairs = (n_chunks + 1) // 2
    last = per_w - CHUNK

    mesh = plsc.VectorSubcoreMesh(core_axis_name="c", subcore_axis_name="s")

    @functools.partial(
        pl.kernel,
        out_type=jax.ShapeDtypeStruct((T, D), jnp.float32),
        mesh=mesh,
        scratch_types=[
            pltpu.VMEM((per_w,), jnp.int32),
            pltpu.VMEM((CHUNK, D), jnp.float32),
            pltpu.VMEM((CHUNK, D), jnp.float32),
            pltpu.SemaphoreType.DMA,
            pltpu.SemaphoreType.DMA,
        ],
        compiler_params=pltpu.CompilerParams(use_tc_tiling_on_sc=False),
    )
    def k(idx_hbm, table_hbm, out_hbm, idx_all, ra, rb, sem_a, sem_b):
        wid = lax.axis_index("s") * NC + lax.axis_index("c")
        base = wid * per_w
        pltpu.sync_copy(idx_hbm.at[pl.ds(base, per_w)], idx_all)

        def off(kk):
            return jnp.minimum(kk * CHUNK, last)

        def start(o, buf, sem):
            pltpu.async_copy(table_hbm.at[idx_all.at[pl.ds(o, CHUNK)]],
                             buf, sem)

        def wait(buf, sem):
            pltpu.make_async_copy(table_hbm.at[idx_all.at[pl.ds(0, CHUNK)]],
                                  buf, sem).wait()

        start(off(0), ra, sem_a)

        def body(k2, carry):
            o1 = off(2 * k2 + 1)
            start(o1, rb, sem_b)
            wait(ra, sem_a)
            pltpu.sync_copy(ra, out_hbm.at[pl.ds(base + off(2 * k2), CHUNK)])
            start(off(2 * k2 + 2), ra, sem_a)
            wait(rb, sem_b)
            pltpu.sync_copy(rb, out_hbm.at[pl.ds(base + o1, CHUNK)])
            return carry

        lax.fori_loop(0, n_pairs, body, 0)
        wait(ra, sem_a)

    return k(idx_flat, table)


def _norm2d(x2, eps=EPS):
    return jnp.sqrt(jnp.maximum(x2, eps))


def _tc_body(gref, hes, hev,
             m1_whT, m1_wsT, m1_b, m1_wv,
             m2_wsT, m2_b, m3_wsT, m3_b, scal,
             ln0_g, ln0_b,
             f1_whT, f1_wsT, f1_b, f1_wvT,
             f2_whT, f2_wsT, f2_b, f2_wvT,
             ln1_g, ln1_b,
             xs_out, xv_out):
    g = gref[...]
    s_i = g[:, 0:32]
    s_j = g[:, 64:96]
    he_s = hes[...]
    he_v = hev[...]
    vex = he_v[:, 0:1]
    vey = he_v[:, 1:2]
    vez = he_v[:, 2:3]

    vx = jnp.concatenate([g[:, 32:36], vex, g[:, 96:100]], axis=1)
    vy = jnp.concatenate([g[:, 36:40], vey, g[:, 100:104]], axis=1)
    vz = jnp.concatenate([g[:, 40:44], vez, g[:, 104:108]], axis=1)

    f32 = jnp.float32

    def dot(a, b):
        return jax.lax.dot_general(a, b, (((1,), (0,)), ((), ())),
                                   preferred_element_type=f32)

    w = m1_whT[...]
    vhx = dot(vx, w)
    vhy = dot(vy, w)
    vhz = dot(vz, w)
    vn = _norm2d(vhx * vhx + vhy * vhy + vhz * vhz)
    s_cat = jnp.concatenate([s_i, he_s, s_j, vn], axis=1)
    s = dot(s_cat, m1_wsT[...]) + m1_b[...]
    wv = m1_wv[...]
    vox = jnp.sum(vhx * wv, axis=1, keepdims=True)
    voy = jnp.sum(vhy * wv, axis=1, keepdims=True)
    voz = jnp.sum(vhz * wv, axis=1, keepdims=True)
    gate = jax.nn.sigmoid(_norm2d(vox * vox + voy * voy + voz * voz))
    vox, voy, voz = vox * gate, voy * gate, voz * gate
    s = jnp.maximum(s, 0.0)

    m2_wh = scal[0, 0]
    m2_wv = scal[0, 1]
    vhx, vhy, vhz = vox * m2_wh, voy * m2_wh, voz * m2_wh
    vn = _norm2d(vhx * vhx + vhy * vhy + vhz * vhz)
    s = dot(jnp.concatenate([s, vn], axis=1), m2_wsT[...]) + m2_b[...]
    vox, voy, voz = vhx * m2_wv, vhy * m2_wv, vhz * m2_wv
    gate = jax.nn.sigmoid(_norm2d(vox * vox + voy * voy + voz * voz))
    vox, voy, voz = vox * gate, voy * gate, voz * gate
    s = jnp.maximum(s, 0.0)

    m3_wh = scal[0, 2]
    m3_wv = scal[0, 3]
    vhx, vhy, vhz = vox * m3_wh, voy * m3_wh, voz * m3_wh
    vn = _norm2d(vhx * vhx + vhy * vhy + vhz * vhz)
    s = dot(jnp.concatenate([s, vn], axis=1), m3_wsT[...]) + m3_b[...]
    vox, voy, voz = vhx * m3_wv, vhy * m3_wv, vhz * m3_wv

    xs = he_s + s
    xvx, xvy, xvz = vex + vox, vey + voy, vez + voz

    def gvp_ln(xs, xvx, xvy, xvz, g_, b_):
        vn2 = jnp.maximum(xvx * xvx + xvy * xvy + xvz * xvz, EPS)
        denom = jnp.sqrt(vn2)
        xvx, xvy, xvz = xvx / denom, xvy / denom, xvz / denom
        mu = jnp.mean(xs, axis=1, keepdims=True)
        var = jnp.mean(jnp.square(xs - mu), axis=1, keepdims=True)
        xs = (xs - mu) / jnp.sqrt(var + 1e-5) * g_[...] + b_[...]
        return xs, xvx, xvy, xvz

    xs, xvx, xvy, xvz = gvp_ln(xs, xvx, xvy, xvz, ln0_g, ln0_b)

    whT = f1_whT[...]
    vhx = xvx * whT
    vhy = xvy * whT
    vhz = xvz * whT
    vn = _norm2d(vhx * vhx + vhy * vhy + vhz * vhz)
    ds = dot(jnp.concatenate([xs, vn], axis=1), f1_wsT[...]) + f1_b[...]
    wvT = f1_wvT[...]
    vox = vhx[:, 0:1] * wvT[0:1, :] + vhx[:, 1:2] * wvT[1:2, :]
    voy = vhy[:, 0:1] * wvT[0:1, :] + vhy[:, 1:2] * wvT[1:2, :]
    voz = vhz[:, 0:1] * wvT[0:1, :] + vhz[:, 1:2] * wvT[1:2, :]
    gate = jax.nn.sigmoid(_norm2d(vox * vox + voy * voy + voz * voz))
    vox, voy, voz = vox * gate, voy * gate, voz * gate
    ds = jnp.maximum(ds, 0.0)

    whT = f2_whT[...]
    vhx = vox[:, 0:1] * whT[0:1, :] + vox[:, 1:2] * whT[1:2, :]
    vhy = voy[:, 0:1] * whT[0:1, :] + voy[:, 1:2] * whT[1:2, :]
    vhz = voz[:, 0:1] * whT[0:1, :] + voz[:, 1:2] * whT[1:2, :]
    vn = _norm2d(vhx * vhx + vhy * vhy + vhz * vhz)
    ds = dot(jnp.concatenate([ds, vn], axis=1), f2_wsT[...]) + f2_b[...]
    wvT = f2_wvT[...]
    vox = vhx[:, 0:1] * wvT[0, 0] + vhx[:, 1:2] * wvT[1, 0]
    voy = vhy[:, 0:1] * wvT[0, 0] + vhy[:, 1:2] * wvT[1, 0]
    voz = vhz[:, 0:1] * wvT[0, 0] + vhz[:, 1:2] * wvT[1, 0]

    xs = xs + ds
    xvx, xvy, xvz = xvx + vox, xvy + voy, xvz + voz
    xs, xvx, xvy, xvz = gvp_ln(xs, xvx, xvy, xvz, ln1_g, ln1_b)

    xs_out[...] = xs
    xv_out[...] = jnp.concatenate([xvx, xvy, xvz], axis=1)


def kernel(h_V_s, h_V_v, edge_index, h_E_s, h_E_v, params):
    p = params
    N, si = h_V_s.shape
    E = edge_index.shape[1]
    f32 = jnp.float32

    table = jnp.concatenate(
        [h_V_s, h_V_v[:, :, 0], h_V_v[:, :, 1], h_V_v[:, :, 2],
         jnp.zeros((N, D - si - 12), f32)], axis=1)

    idx_flat = edge_index.T.reshape(-1)
    g_all = _sc_gather(table, idx_flat)
    g2 = g_all.reshape(E, 2 * D)

    hev = h_E_v.reshape(E, 3)

    n_blk = E // BLK
    assert n_blk * BLK == E

    def bspec(shape):
        return pl.BlockSpec(shape, lambda i: (i, 0))

    def wspec(a):
        return pl.BlockSpec(a.shape, lambda i: tuple(0 for _ in a.shape))

    weights = [
        p['m1_wh'].T, p['m1_ws_w'].T, p['m1_ws_b'].reshape(1, 32), p['m1_wv'],
        p['m2_ws_w'].T, p['m2_ws_b'].reshape(1, 32),
        p['m3_ws_w'].T, p['m3_ws_b'].reshape(1, 32),
        jnp.stack([p['m2_wh'][0, 0], p['m2_wv'][0, 0],
                   p['m3_wh'][0, 0], p['m3_wv'][0, 0]]).reshape(1, 4),
        p['ln0_g'].reshape(1, 32), p['ln0_b'].reshape(1, 32),
        p['f1_wh'].T, p['f1_ws_w'].T, p['f1_ws_b'].reshape(1, 128), p['f1_wv'].T,
        p['f2_wh'].T, p['f2_ws_w'].T, p['f2_ws_b'].reshape(1, 32), p['f2_wv'].T,
        p['ln1_g'].reshape(1, 32), p['ln1_b'].reshape(1, 32),
    ]

    in_specs = ([bspec((BLK, 2 * D)), bspec((BLK, si)), bspec((BLK, 3))]
                + [wspec(w) for w in weights])

    xs, xv = pl.pallas_call(
        _tc_body,
        grid=(n_blk,),
        in_specs=in_specs,
        out_specs=[bspec((BLK, si)), bspec((BLK, 3))],
        out_shape=[jax.ShapeDtypeStruct((E, si), f32),
                   jax.ShapeDtypeStruct((E, 3), f32)],
    )(g2, h_E_s, hev, *weights)

    return xs, xv.reshape(E, 1, 3)


# speedup vs baseline: 9.3669x; 9.2876x over previous
import functools

import jax
import jax.numpy as jnp
from jax import lax
from jax.experimental import pallas as pl
from jax.experimental.pallas import tpu as pltpu
from jax.experimental.pallas import tpu_sc as plsc

NC = 2
NS = 16
NW = NC * NS
CHUNK = 128
D = 64
EPS = 1e-8

BLK = 3200


def _sc_gather(table, src_idx, dst_idx):
    E = src_idx.shape[0]
    per_w = E // NW
    assert per_w * NW == E and per_w % 8 == 0 and (per_w - CHUNK) % 8 == 0
    n_chunks = (per_w + CHUNK - 1) // CHUNK
    last = per_w - CHUNK

    mesh = plsc.VectorSubcoreMesh(core_axis_name="c", subcore_axis_name="s")

    @functools.partial(
        pl.kernel,
        out_type=jax.ShapeDtypeStruct((E, 2 * D), jnp.float32),
        mesh=mesh,
        scratch_types=[
            pltpu.VMEM((per_w,), jnp.int32),
            pltpu.VMEM((per_w,), jnp.int32),
            pltpu.VMEM((CHUNK, D), jnp.float32),
            pltpu.VMEM((CHUNK, D), jnp.float32),
            pltpu.SemaphoreType.DMA,
            pltpu.SemaphoreType.DMA,
        ],
        compiler_params=pltpu.CompilerParams(use_tc_tiling_on_sc=False),
    )
    def k(src_hbm, dst_hbm, table_hbm, out_hbm, si_v, di_v, ra, rb,
          sem_a, sem_b):
        wid = lax.axis_index("s") * NC + lax.axis_index("c")
        base = wid * per_w
        pltpu.sync_copy(src_hbm.at[pl.ds(base, per_w)], si_v)
        pltpu.sync_copy(dst_hbm.at[pl.ds(base, per_w)], di_v)

        def off(kk):
            return jnp.minimum(kk * CHUNK, last)

        def start(idx_v, o, buf, sem):
            pltpu.async_copy(table_hbm.at[idx_v.at[pl.ds(o, CHUNK)]],
                             buf, sem)

        def wait(buf, sem):
            pltpu.make_async_copy(table_hbm.at[si_v.at[pl.ds(0, CHUNK)]],
                                  buf, sem).wait()

        start(si_v, 0, ra, sem_a)

        def body(kk, carry):
            o = off(kk)
            start(di_v, o, rb, sem_b)
            wait(ra, sem_a)
            pltpu.sync_copy(
                ra, out_hbm.at[pl.ds(base + o, CHUNK), pl.ds(0, D)])
            o1 = off(kk + 1)
            start(si_v, o1, ra, sem_a)
            wait(rb, sem_b)
            pltpu.sync_copy(
                rb, out_hbm.at[pl.ds(base + o, CHUNK), pl.ds(D, D)])
            return carry

        lax.fori_loop(0, n_chunks, body, 0)
        wait(ra, sem_a)

    return k(src_idx, dst_idx, table)


def _sqnorm(parts):
    acc = parts[0] * parts[0]
    for q in parts[1:]:
        acc = acc + q * q
    return acc


def _norm(parts):
    return jnp.sqrt(jnp.maximum(_sqnorm(parts), EPS))


def _tc_body(gref, hesT, hevT,
             m1_wh, m1_ws, m1_b, m1_wv,
             m2_ws, m2_b, m3_ws, m3_b, scal,
             ln0_g, ln0_b,
             f1_wh, f1_ws, f1_b, f1_wv,
             f2_wh, f2_ws, f2_b, f2_wv,
             ln1_g, ln1_b,
             xs_out, xv_out):
    bf16 = jnp.bfloat16

    def dot(w_ref, x):
        return jax.lax.dot_general(w_ref[...], x.astype(bf16),
                                   (((1,), (0,)), ((), ())),
                                   preferred_element_type=jnp.float32)

    gT = jnp.transpose(gref[...], (1, 0))
    s_i = gT[0:32]
    s_j = gT[64:96]
    he_s = hesT[...]
    he_v = hevT[...]
    vex = he_v[0:1]
    vey = he_v[1:2]
    vez = he_v[2:3]

    vx = jnp.concatenate([gT[32:36], vex, gT[96:100]], axis=0)
    vy = jnp.concatenate([gT[36:40], vey, gT[100:104]], axis=0)
    vz = jnp.concatenate([gT[40:44], vez, gT[104:108]], axis=0)

    vhx = dot(m1_wh, vx)
    vhy = dot(m1_wh, vy)
    vhz = dot(m1_wh, vz)
    vn = _norm([vhx, vhy, vhz])
    x_cat = jnp.concatenate([s_i, he_s, s_j, vn], axis=0)
    s = dot(m1_ws, x_cat) + m1_b[...]
    wv = m1_wv[...]
    vox = jnp.sum(vhx * wv, axis=0, keepdims=True)
    voy = jnp.sum(vhy * wv, axis=0, keepdims=True)
    voz = jnp.sum(vhz * wv, axis=0, keepdims=True)
    gate = jax.nn.sigmoid(_norm([vox, voy, voz]))
    vox, voy, voz = vox * gate, voy * gate, voz * gate
    s = jnp.maximum(s, 0.0)

    m2_wh = scal[0, 0]
    m2_wv = scal[0, 1]
    vhx, vhy, vhz = vox * m2_wh, voy * m2_wh, voz * m2_wh
    vn = _norm([vhx, vhy, vhz])
    s = dot(m2_ws, jnp.concatenate([s, vn], axis=0)) + m2_b[...]
    vox, voy, voz = vhx * m2_wv, vhy * m2_wv, vhz * m2_wv
    gate = jax.nn.sigmoid(_norm([vox, voy, voz]))
    vox, voy, voz = vox * gate, voy * gate, voz * gate
    s = jnp.maximum(s, 0.0)

    m3_wh = scal[0, 2]
    m3_wv = scal[0, 3]
    vhx, vhy, vhz = vox * m3_wh, voy * m3_wh, voz * m3_wh
    vn = _norm([vhx, vhy, vhz])
    s = dot(m3_ws, jnp.concatenate([s, vn], axis=0)) + m3_b[...]
    vox, voy, voz = vhx * m3_wv, vhy * m3_wv, vhz * m3_wv

    xs = he_s + s
    xvx, xvy, xvz = vex + vox, vey + voy, vez + voz

    def gvp_ln(xs, xvx, xvy, xvz, g_, b_):
        rn = jax.lax.rsqrt(jnp.maximum(_sqnorm([xvx, xvy, xvz]), EPS))
        xvx, xvy, xvz = xvx * rn, xvy * rn, xvz * rn
        mu = jnp.mean(xs, axis=0, keepdims=True)
        var = jnp.mean(jnp.square(xs - mu), axis=0, keepdims=True)
        xs = (xs - mu) * jax.lax.rsqrt(var + 1e-5) * g_[...] + b_[...]
        return xs, xvx, xvy, xvz

    xs, xvx, xvy, xvz = gvp_ln(xs, xvx, xvy, xvz, ln0_g, ln0_b)

    w0 = f1_wh[0, 0]
    w1 = f1_wh[1, 0]
    vhx = jnp.concatenate([xvx * w0, xvx * w1], axis=0)
    vhy = jnp.concatenate([xvy * w0, xvy * w1], axis=0)
    vhz = jnp.concatenate([xvz * w0, xvz * w1], axis=0)
    vn = _norm([vhx, vhy, vhz])
    ds = dot(f1_ws, jnp.concatenate([xs, vn], axis=0)) + f1_b[...]

    def mat2(w_ref, ax, ay, az, r, h0, h1):
        a = w_ref[r, h0]
        b = w_ref[r, h1]
        return (ax[h0:h0 + 1] * a + ax[h1:h1 + 1] * b,
                ay[h0:h0 + 1] * a + ay[h1:h1 + 1] * b,
                az[h0:h0 + 1] * a + az[h1:h1 + 1] * b)

    o0 = mat2(f1_wv, vhx, vhy, vhz, 0, 0, 1)
    o1 = mat2(f1_wv, vhx, vhy, vhz, 1, 0, 1)
    gate0 = jax.nn.sigmoid(_norm([o0[0], o0[1], o0[2]]))
    gate1 = jax.nn.sigmoid(_norm([o1[0], o1[1], o1[2]]))
    vox = jnp.concatenate([o0[0] * gate0, o1[0] * gate1], axis=0)
    voy = jnp.concatenate([o0[1] * gate0, o1[1] * gate1], axis=0)
    voz = jnp.concatenate([o0[2] * gate0, o1[2] * gate1], axis=0)
    ds = jnp.maximum(ds, 0.0)

    h0 = mat2(f2_wh, vox, voy, voz, 0, 0, 1)
    h1 = mat2(f2_wh, vox, voy, voz, 1, 0, 1)
    vhx = jnp.concatenate([h0[0], h1[0]], axis=0)
    vhy = jnp.concatenate([h0[1], h1[1]], axis=0)
    vhz = jnp.concatenate([h0[2], h1[2]], axis=0)
    vn = _norm([vhx, vhy, vhz])
    ds = dot(f2_ws, jnp.concatenate([ds, vn], axis=0)) + f2_b[...]
    vox, voy, voz = mat2(f2_wv, vhx, vhy, vhz, 0, 0, 1)

    xs = xs + ds
    xvx, xvy, xvz = xvx + vox, xvy + voy, xvz + voz
    xs, xvx, xvy, xvz = gvp_ln(xs, xvx, xvy, xvz, ln1_g, ln1_b)

    xs_out[...] = xs
    xv_out[...] = jnp.concatenate([xvx, xvy, xvz], axis=0)


def kernel(h_V_s, h_V_v, edge_index, h_E_s, h_E_v, params):
    p = params
    N, si = h_V_s.shape
    E = edge_index.shape[1]
    f32 = jnp.float32
    bf16 = jnp.bfloat16

    table = jnp.concatenate(
        [h_V_s, h_V_v[:, :, 0], h_V_v[:, :, 1], h_V_v[:, :, 2],
         jnp.zeros((N, D - si - 12), f32)], axis=1)

    g2 = _sc_gather(table, edge_index[0], edge_index[1])

    hesT = h_E_s.T
    hevT = h_E_v.reshape(E, 3).T

    n_blk = E // BLK
    assert n_blk * BLK == E

    def fspec(c):
        return pl.BlockSpec((c, BLK), lambda i: (0, i))

    def wspec(a):
        return pl.BlockSpec(a.shape, lambda i: tuple(0 for _ in a.shape))

    weights = [
        p['m1_wh'].astype(bf16), p['m1_ws_w'].astype(bf16),
        p['m1_ws_b'].reshape(32, 1), p['m1_wv'].reshape(9, 1),
        p['m2_ws_w'].astype(bf16), p['m2_ws_b'].reshape(32, 1),
        p['m3_ws_w'].astype(bf16), p['m3_ws_b'].reshape(32, 1),
        jnp.stack([p['m2_wh'][0, 0], p['m2_wv'][0, 0],
                   p['m3_wh'][0, 0], p['m3_wv'][0, 0]]).reshape(1, 4),
        p['ln0_g'].reshape(32, 1), p['ln0_b'].reshape(32, 1),
        p['f1_wh'], p['f1_ws_w'].astype(bf16),
        p['f1_ws_b'].reshape(128, 1), p['f1_wv'],
        p['f2_wh'], p['f2_ws_w'].astype(bf16),
        p['f2_ws_b'].reshape(32, 1), p['f2_wv'],
        p['ln1_g'].reshape(32, 1), p['ln1_b'].reshape(32, 1),
    ]

    in_specs = ([pl.BlockSpec((BLK, 2 * D), lambda i: (i, 0)),
                 fspec(si), fspec(3)]
                + [wspec(w) for w in weights])

    xs_fm, xv_fm = pl.pallas_call(
        _tc_body,
        grid=(n_blk,),
        in_specs=in_specs,
        out_specs=[fspec(si), fspec(3)],
        out_shape=[jax.ShapeDtypeStruct((si, E), f32),
                   jax.ShapeDtypeStruct((3, E), f32)],
    )(g2, hesT, hevT, *weights)

    return xs_fm.T, xv_fm.T.reshape(E, 1, 3)


# speedup vs baseline: 12.0735x; 1.2889x over previous
import functools

import jax
import jax.numpy as jnp
from jax import lax
from jax.experimental import pallas as pl
from jax.experimental.pallas import tpu as pltpu
from jax.experimental.pallas import tpu_sc as plsc

NC = 2
NS = 16
NW = NC * NS
CHUNK = 128
D = 64
EPS = 1e-8

BLK = 6400
NSLICE = 5


def _sc_gather(table, src_idx, dst_idx):
    E = src_idx.shape[0]
    per_w = E // NW
    assert per_w * NW == E and per_w % 8 == 0 and (per_w - CHUNK) % 8 == 0
    n_chunks = (per_w + CHUNK - 1) // CHUNK
    last = per_w - CHUNK

    mesh = plsc.VectorSubcoreMesh(core_axis_name="c", subcore_axis_name="s")

    @functools.partial(
        pl.kernel,
        out_type=jax.ShapeDtypeStruct((E, 2 * D), jnp.float32),
        mesh=mesh,
        scratch_types=[
            pltpu.VMEM((per_w,), jnp.int32),
            pltpu.VMEM((per_w,), jnp.int32),
            pltpu.VMEM((CHUNK, D), jnp.float32),
            pltpu.VMEM((CHUNK, D), jnp.float32),
            pltpu.SemaphoreType.DMA,
            pltpu.SemaphoreType.DMA,
        ],
        compiler_params=pltpu.CompilerParams(use_tc_tiling_on_sc=False),
    )
    def k(src_hbm, dst_hbm, table_hbm, out_hbm, si_v, di_v, ra, rb,
          sem_a, sem_b):
        wid = lax.axis_index("s") * NC + lax.axis_index("c")
        base = wid * per_w
        pltpu.sync_copy(src_hbm.at[pl.ds(base, per_w)], si_v)
        pltpu.sync_copy(dst_hbm.at[pl.ds(base, per_w)], di_v)

        def off(kk):
            return jnp.minimum(kk * CHUNK, last)

        def start(idx_v, o, buf, sem):
            pltpu.async_copy(table_hbm.at[idx_v.at[pl.ds(o, CHUNK)]],
                             buf, sem)

        def wait(buf, sem):
            pltpu.make_async_copy(table_hbm.at[si_v.at[pl.ds(0, CHUNK)]],
                                  buf, sem).wait()

        start(si_v, 0, ra, sem_a)

        def body(kk, carry):
            o = off(kk)
            start(di_v, o, rb, sem_b)
            wait(ra, sem_a)
            pltpu.sync_copy(
                ra, out_hbm.at[pl.ds(base + o, CHUNK), pl.ds(0, D)])
            o1 = off(kk + 1)
            start(si_v, o1, ra, sem_a)
            wait(rb, sem_b)
            pltpu.sync_copy(
                rb, out_hbm.at[pl.ds(base + o, CHUNK), pl.ds(D, D)])
            return carry

        lax.fori_loop(0, n_chunks, body, 0)
        wait(ra, sem_a)

    return k(src_idx, dst_idx, table)


def _sqn(x, y, z):
    return x * x + y * y + z * z


def _norm3(x, y, z):
    return jnp.sqrt(jnp.maximum(_sqn(x, y, z), EPS))


def _tc_body(gref, hesT, hevT,
             w_all, ws_hevn, m1_b, m1_wv, wh_e,
             m2_ws, m2_vnc, m2_b, m3_ws, m3_vnc, m3_b, scal,
             ln0_g, ln0_b, w_mean,
             f1_ws, f1_b,
             f2_ws, f2_b,
             ln1_g, ln1_b,
             xs_out, xv_out):
    bf16 = jnp.bfloat16

    def dotw(w_ref, x):
        return jax.lax.dot_general(w_ref[...], x,
                                   (((1,), (0,)), ((), ())),
                                   preferred_element_type=jnp.float32)

    def k(i):
        return scal[0, i]

    he_s = hesT[...]
    he_v = hevT[...]
    vex = he_v[0:1]
    vey = he_v[1:2]
    vez = he_v[2:3]

    out = jax.lax.dot_general(w_all[...], gref[...].astype(bf16),
                              (((1,), (1,)), ((), ())),
                              preferred_element_type=jnp.float32)
    whe = wh_e[...]
    vhx = out[32:41] + whe * vex
    vhy = out[48:57] + whe * vey
    vhz = out[64:73] + whe * vez
    vn = _norm3(vhx, vhy, vhz)
    he_vn = jnp.concatenate([he_s.astype(bf16), vn.astype(bf16)], axis=0)
    s = out[0:32] + dotw(ws_hevn, he_vn) + m1_b[...]
    wv = m1_wv[...]
    vox = jnp.sum(vhx * wv, axis=0, keepdims=True)
    voy = jnp.sum(vhy * wv, axis=0, keepdims=True)
    voz = jnp.sum(vhz * wv, axis=0, keepdims=True)
    gate = jax.nn.sigmoid(_norm3(vox, voy, voz))
    vox, voy, voz = vox * gate, voy * gate, voz * gate
    s = jnp.maximum(s, 0.0)

    vhx, vhy, vhz = vox * k(0), voy * k(0), voz * k(0)
    vn1 = _norm3(vhx, vhy, vhz)
    s = dotw(m2_ws, s.astype(bf16)) + m2_vnc[...] * vn1 + m2_b[...]
    vox, voy, voz = vhx * k(1), vhy * k(1), vhz * k(1)
    gate = jax.nn.sigmoid(_norm3(vox, voy, voz))
    vox, voy, voz = vox * gate, voy * gate, voz * gate
    s = jnp.maximum(s, 0.0)

    vhx, vhy, vhz = vox * k(2), voy * k(2), voz * k(2)
    vn1 = _norm3(vhx, vhy, vhz)
    s = dotw(m3_ws, s.astype(bf16)) + m3_vnc[...] * vn1 + m3_b[...]
    vox, voy, voz = vhx * k(3), vhy * k(3), vhz * k(3)

    xs = he_s + s
    xvx, xvy, xvz = vex + vox, vey + voy, vez + voz

    def gvp_ln(xs, xvx, xvy, xvz, g_, b_):
        rn = jax.lax.rsqrt(jnp.maximum(_sqn(xvx, xvy, xvz), EPS))
        xvx, xvy, xvz = xvx * rn, xvy * rn, xvz * rn
        xb = xs.astype(bf16)
        mu = dotw(w_mean, xb)
        ex2 = dotw(w_mean, (xs * xs).astype(bf16))
        var = ex2 - mu * mu
        xs = (xs - mu) * jax.lax.rsqrt(var + 1e-5) * g_[...] + b_[...]
        return xs, xvx, xvy, xvz

    xs, xvx, xvy, xvz = gvp_ln(xs, xvx, xvy, xvz, ln0_g, ln0_b)

    h0x, h0y, h0z = xvx * k(4), xvy * k(4), xvz * k(4)
    h1x, h1y, h1z = xvx * k(5), xvy * k(5), xvz * k(5)
    vn0 = _norm3(h0x, h0y, h0z)
    vn1 = _norm3(h1x, h1y, h1z)
    x34 = jnp.concatenate([xs.astype(bf16), vn0.astype(bf16),
                           vn1.astype(bf16)], axis=0)
    ds = dotw(f1_ws, x34).astype(bf16) + f1_b[...]
    o0x = k(6) * h0x + k(7) * h1x
    o0y = k(6) * h0y + k(7) * h1y
    o0z = k(6) * h0z + k(7) * h1z
    o1x = k(8) * h0x + k(9) * h1x
    o1y = k(8) * h0y + k(9) * h1y
    o1z = k(8) * h0z + k(9) * h1z
    g0 = jax.nn.sigmoid(_norm3(o0x, o0y, o0z))
    g1 = jax.nn.sigmoid(_norm3(o1x, o1y, o1z))
    o0x, o0y, o0z = o0x * g0, o0y * g0, o0z * g0
    o1x, o1y, o1z = o1x * g1, o1y * g1, o1z * g1
    ds = jnp.maximum(ds, jnp.zeros((), bf16))

    h0x = k(10) * o0x + k(11) * o1x
    h0y = k(10) * o0y + k(11) * o1y
    h0z = k(10) * o0z + k(11) * o1z
    h1x = k(12) * o0x + k(13) * o1x
    h1y = k(12) * o0y + k(13) * o1y
    h1z = k(12) * o0z + k(13) * o1z
    vn0 = _norm3(h0x, h0y, h0z)
    vn1 = _norm3(h1x, h1y, h1z)
    x130 = jnp.concatenate([ds, vn0.astype(bf16), vn1.astype(bf16)],
                           axis=0)
    ds = dotw(f2_ws, x130) + f2_b[...]
    vox = k(14) * h0x + k(15) * h1x
    voy = k(14) * h0y + k(15) * h1y
    voz = k(14) * h0z + k(15) * h1z

    xs = xs + ds
    xvx, xvy, xvz = xvx + vox, xvy + voy, xvz + voz
    xs, xvx, xvy, xvz = gvp_ln(xs, xvx, xvy, xvz, ln1_g, ln1_b)

    xs_out[...] = xs
    xv_out[...] = jnp.concatenate([xvx, xvy, xvz], axis=0)


def _build_weights(p):
    f32 = jnp.float32
    bf16 = jnp.bfloat16
    m1_ws = p['m1_ws_w']
    m1_wh = p['m1_wh']
    w_all = jnp.zeros((80, 128), f32)
    w_all = w_all.at[0:32, 0:32].set(m1_ws[:, 0:32])
    w_all = w_all.at[0:32, 64:96].set(m1_ws[:, 64:96])
    for comp, (r0, c_i, c_j) in enumerate([(32, 32, 96), (48, 36, 100),
                                           (64, 40, 104)]):
        w_all = w_all.at[r0:r0 + 9, c_i:c_i + 4].set(m1_wh[:, 0:4])
        w_all = w_all.at[r0:r0 + 9, c_j:c_j + 4].set(m1_wh[:, 5:9])
    scal = jnp.stack([
        p['m2_wh'][0, 0], p['m2_wv'][0, 0],
        p['m3_wh'][0, 0], p['m3_wv'][0, 0],
        p['f1_wh'][0, 0], p['f1_wh'][1, 0],
        p['f1_wv'][0, 0], p['f1_wv'][0, 1],
        p['f1_wv'][1, 0], p['f1_wv'][1, 1],
        p['f2_wh'][0, 0], p['f2_wh'][0, 1],
        p['f2_wh'][1, 0], p['f2_wh'][1, 1],
        p['f2_wv'][0, 0], p['f2_wv'][0, 1],
    ]).reshape(1, 16)
    return [
        w_all.astype(bf16),
        jnp.concatenate([m1_ws[:, 32:64], m1_ws[:, 96:105]],
                        axis=1).astype(bf16),
        p['m1_ws_b'].reshape(32, 1),
        p['m1_wv'].reshape(9, 1),
        m1_wh[:, 4].reshape(9, 1),
        p['m2_ws_w'][:, 0:32].astype(bf16),
        p['m2_ws_w'][:, 32:33],
        p['m2_ws_b'].reshape(32, 1),
        p['m3_ws_w'][:, 0:32].astype(bf16),
        p['m3_ws_w'][:, 32:33],
        p['m3_ws_b'].reshape(32, 1),
        scal,
        p['ln0_g'].reshape(32, 1), p['ln0_b'].reshape(32, 1),
        jnp.full((1, 32), 1.0 / 32.0, bf16),
        p['f1_ws_w'].astype(bf16),
        p['f1_ws_b'].reshape(128, 1).astype(bf16),
        p['f2_ws_w'].astype(bf16),
        p['f2_ws_b'].reshape(32, 1),
        p['ln1_g'].reshape(32, 1), p['ln1_b'].reshape(32, 1),
    ]


def kernel(h_V_s, h_V_v, edge_index, h_E_s, h_E_v, params):
    N, si = h_V_s.shape
    E = edge_index.shape[1]
    f32 = jnp.float32

    table = jnp.concatenate(
        [h_V_s, h_V_v[:, :, 0], h_V_v[:, :, 1], h_V_v[:, :, 2],
         jnp.zeros((N, D - si - 12), f32)], axis=1)

    hesT = h_E_s.T
    hevT = h_E_v.reshape(E, 3).T
    weights = _build_weights(params)

    Es = E // NSLICE
    nb = Es // BLK
    assert Es * NSLICE == E and nb * BLK == Es

    def wspec(a):
        return pl.BlockSpec(a.shape, lambda i: tuple(0 for _ in a.shape))

    xs_parts = []
    xv_parts = []
    for sl in range(NSLICE):
        lo = sl * Es
        g_s = _sc_gather(table,
                         lax.slice(edge_index[0], (lo,), (lo + Es,)),
                         lax.slice(edge_index[1], (lo,), (lo + Es,)))

        def espec(c, sl=sl):
            return pl.BlockSpec((c, BLK), lambda i, sl=sl: (0, i + sl * nb))

        in_specs = ([pl.BlockSpec((BLK, 2 * D), lambda i: (i, 0)),
                     espec(si), espec(3)]
                    + [wspec(w) for w in weights])

        xs_fm, xv_fm = pl.pallas_call(
            _tc_body,
            grid=(nb,),
            in_specs=in_specs,
            out_specs=[pl.BlockSpec((si, BLK), lambda i: (0, i)),
                       pl.BlockSpec((3, BLK), lambda i: (0, i))],
            out_shape=[jax.ShapeDtypeStruct((si, Es), f32),
                       jax.ShapeDtypeStruct((3, Es), f32)],
        )(g_s, hesT, hevT, *weights)
        xs_parts.append(xs_fm.T)
        xv_parts.append(xv_fm.T)

    xs = jnp.concatenate(xs_parts, axis=0)
    xv = jnp.concatenate(xv_parts, axis=0)
    return xs, xv.reshape(E, 1, 3)
